# baseline (speedup 1.0000x reference)
import jax
import jax.numpy as jnp
from jax.experimental import pallas as pl
from jax.experimental.pallas import tpu as pltpu

_NUM_EMB = 1024
_DIM = 64
_N = 8 * 576
_R = 288


def _vq_kernel(x_ref, w_ref, q_ref, enc_ref, idx_ref, loss_ref,
               w2_acc, counts_acc, mse_acc):
    i = pl.program_id(0)
    nsteps = pl.num_programs(0)

    xb = x_ref[...]
    w = w_ref[...]

    @pl.when(i == 0)
    def _w2():
        w2_acc[...] = jnp.sum(w * w, axis=1)[None, :]

    x2 = jnp.sum(xb * xb, axis=1, keepdims=True)
    xw = jax.lax.dot_general(xb, w, (((1,), (1,)), ((), ())),
                             preferred_element_type=jnp.float32)
    d2 = x2 + w2_acc[...] - 2.0 * xw

    idx = jnp.argmin(d2, axis=1).astype(jnp.int32)
    idx_ref[0, 0] = idx

    cols = jax.lax.broadcasted_iota(jnp.int32, d2.shape, 1)
    one_hot = (cols == idx[:, None]).astype(jnp.float32)
    enc_ref[...] = one_hot

    q = jax.lax.dot_general(one_hot, w, (((1,), (0,)), ((), ())),
                            preferred_element_type=jnp.float32)
    q_ref[...] = q

    diff = q - xb
    mse_part = jnp.sum(diff * diff).reshape(1, 1)
    counts_part = jnp.sum(one_hot, axis=0, keepdims=True)

    @pl.when(i == 0)
    def _init():
        counts_acc[...] = counts_part
        mse_acc[...] = mse_part

    @pl.when(i != 0)
    def _acc():
        counts_acc[...] += counts_part
        mse_acc[...] += mse_part

    @pl.when(i == nsteps - 1)
    def _finalize():
        usage = counts_acc[...] * (1.0 / _N)
        du = usage - (1.0 / _NUM_EMB)
        usage_loss = jnp.sum(du * du) * (1.0 / _NUM_EMB)
        mse = mse_acc[...] * (1.0 / (_N * _DIM))
        loss_ref[...] = 1.25 * mse + 0.1 * usage_loss


@jax.jit
def kernel(x, W):
    b, l, d = x.shape
    n = b * l
    flat = x.reshape(n, d)
    grid = (n // _R,)
    out_types = (
        jax.ShapeDtypeStruct((n, d), jnp.float32),
        jax.ShapeDtypeStruct((n, _NUM_EMB), jnp.float32),
        jax.ShapeDtypeStruct((n // _R, 1, _R), jnp.int32),
        jax.ShapeDtypeStruct((1, 1), jnp.float32),
    )
    q, enc, idx, loss = pl.pallas_call(
        _vq_kernel,
        grid=grid,
        in_specs=[
            pl.BlockSpec((_R, d), lambda i: (i, 0)),
            pl.BlockSpec((_NUM_EMB, d), lambda i: (0, 0)),
        ],
        out_specs=(
            pl.BlockSpec((_R, d), lambda i: (i, 0)),
            pl.BlockSpec((_R, _NUM_EMB), lambda i: (i, 0)),
            pl.BlockSpec((1, 1, _R), lambda i: (i, 0, 0)),
            pl.BlockSpec((1, 1), lambda i: (0, 0)),
        ),
        out_shape=out_types,
        scratch_shapes=[
            pltpu.VMEM((1, _NUM_EMB), jnp.float32),
            pltpu.VMEM((1, _NUM_EMB), jnp.float32),
            pltpu.VMEM((1, 1), jnp.float32),
        ],
    )(flat, W)
    return (q.reshape(b, l, d), loss.reshape(()),
            enc.reshape(b, l, _NUM_EMB), idx.reshape(b, l))


# speedup vs baseline: 1.1071x; 1.1071x over previous
import jax
import jax.numpy as jnp
from jax.experimental import pallas as pl
from jax.experimental.pallas import tpu as pltpu

_NUM_EMB = 1024
_DIM = 64
_N = 8 * 576
_R = 1152


def _vq_kernel(x_ref, w_ref, q_ref, enc_ref, idx_ref, loss_ref,
               w2_acc, counts_acc, mse_acc):
    i = pl.program_id(0)
    nsteps = pl.num_programs(0)

    xb = x_ref[...]
    w = w_ref[...]

    @pl.when(i == 0)
    def _w2():
        w2_acc[...] = jnp.sum(w * w, axis=1)[None, :]

    x2 = jnp.sum(xb * xb, axis=1, keepdims=True)
    xw = jax.lax.dot_general(xb, w, (((1,), (1,)), ((), ())),
                             preferred_element_type=jnp.float32)
    d2 = x2 + w2_acc[...] - 2.0 * xw

    idx = jnp.argmin(d2, axis=1).astype(jnp.int32)
    idx_ref[0, 0] = idx

    cols = jax.lax.broadcasted_iota(jnp.int32, d2.shape, 1)
    one_hot = (cols == idx[:, None]).astype(jnp.float32)
    enc_ref[...] = one_hot

    q = jax.lax.dot_general(one_hot, w, (((1,), (0,)), ((), ())),
                            preferred_element_type=jnp.float32)
    q_ref[...] = q

    diff = q - xb
    mse_part = jnp.sum(diff * diff).reshape(1, 1)
    counts_part = jnp.sum(one_hot, axis=0, keepdims=True)

    @pl.when(i == 0)
    def _init():
        counts_acc[...] = counts_part
        mse_acc[...] = mse_part

    @pl.when(i != 0)
    def _acc():
        counts_acc[...] += counts_part
        mse_acc[...] += mse_part

    @pl.when(i == nsteps - 1)
    def _finalize():
        usage = counts_acc[...] * (1.0 / _N)
        du = usage - (1.0 / _NUM_EMB)
        usage_loss = jnp.sum(du * du) * (1.0 / _NUM_EMB)
        mse = mse_acc[...] * (1.0 / (_N * _DIM))
        loss_ref[...] = 1.25 * mse + 0.1 * usage_loss


@jax.jit
def kernel(x, W):
    b, l, d = x.shape
    n = b * l
    flat = x.reshape(n, d)
    grid = (n // _R,)
    out_types = (
        jax.ShapeDtypeStruct((n, d), jnp.float32),
        jax.ShapeDtypeStruct((n, _NUM_EMB), jnp.float32),
        jax.ShapeDtypeStruct((n // _R, 1, _R), jnp.int32),
        jax.ShapeDtypeStruct((1, 1), jnp.float32),
    )
    q, enc, idx, loss = pl.pallas_call(
        _vq_kernel,
        grid=grid,
        in_specs=[
            pl.BlockSpec((_R, d), lambda i: (i, 0)),
            pl.BlockSpec((_NUM_EMB, d), lambda i: (0, 0)),
        ],
        out_specs=(
            pl.BlockSpec((_R, d), lambda i: (i, 0)),
            pl.BlockSpec((_R, _NUM_EMB), lambda i: (i, 0)),
            pl.BlockSpec((1, 1, _R), lambda i: (i, 0, 0)),
            pl.BlockSpec((1, 1), lambda i: (0, 0)),
        ),
        out_shape=out_types,
        scratch_shapes=[
            pltpu.VMEM((1, _NUM_EMB), jnp.float32),
            pltpu.VMEM((1, _NUM_EMB), jnp.float32),
            pltpu.VMEM((1, 1), jnp.float32),
        ],
    )(flat, W)
    return (q.reshape(b, l, d), loss.reshape(()),
            enc.reshape(b, l, _NUM_EMB), idx.reshape(b, l))


# speedup vs baseline: 1.1456x; 1.0348x over previous
import jax
import jax.numpy as jnp
from jax.experimental import pallas as pl
from jax.experimental.pallas import tpu as pltpu

_NUM_EMB = 1024
_DIM = 64
_N = 8 * 576
_R = 2304


def _vq_kernel(x_ref, w_ref, q_ref, enc_ref, idx_ref, loss_ref,
               w2_acc, counts_acc, mse_acc):
    i = pl.program_id(0)
    nsteps = pl.num_programs(0)

    xb = x_ref[...]
    w = w_ref[...]

    @pl.when(i == 0)
    def _w2():
        w2_acc[...] = jnp.sum(w * w, axis=1)[None, :]

    x2 = jnp.sum(xb * xb, axis=1, keepdims=True)
    xw = jax.lax.dot_general(xb, w, (((1,), (1,)), ((), ())),
                             preferred_element_type=jnp.float32)
    d2 = x2 + w2_acc[...] - 2.0 * xw

    idx = jnp.argmin(d2, axis=1).astype(jnp.int32)
    idx_ref[0, 0] = idx

    cols = jax.lax.broadcasted_iota(jnp.int32, d2.shape, 1)
    one_hot = (cols == idx[:, None]).astype(jnp.float32)
    enc_ref[...] = one_hot

    q = jax.lax.dot_general(one_hot, w, (((1,), (0,)), ((), ())),
                            preferred_element_type=jnp.float32)
    q_ref[...] = q

    diff = q - xb
    mse_part = jnp.sum(diff * diff).reshape(1, 1)
    counts_part = jnp.sum(one_hot, axis=0, keepdims=True)

    @pl.when(i == 0)
    def _init():
        counts_acc[...] = counts_part
        mse_acc[...] = mse_part

    @pl.when(i != 0)
    def _acc():
        counts_acc[...] += counts_part
        mse_acc[...] += mse_part

    @pl.when(i == nsteps - 1)
    def _finalize():
        usage = counts_acc[...] * (1.0 / _N)
        du = usage - (1.0 / _NUM_EMB)
        usage_loss = jnp.sum(du * du) * (1.0 / _NUM_EMB)
        mse = mse_acc[...] * (1.0 / (_N * _DIM))
        loss_ref[...] = 1.25 * mse + 0.1 * usage_loss


@jax.jit
def kernel(x, W):
    b, l, d = x.shape
    n = b * l
    flat = x.reshape(n, d)
    grid = (n // _R,)
    out_types = (
        jax.ShapeDtypeStruct((n, d), jnp.float32),
        jax.ShapeDtypeStruct((n, _NUM_EMB), jnp.float32),
        jax.ShapeDtypeStruct((n // _R, 1, _R), jnp.int32),
        jax.ShapeDtypeStruct((1, 1), jnp.float32),
    )
    q, enc, idx, loss = pl.pallas_call(
        _vq_kernel,
        grid=grid,
        in_specs=[
            pl.BlockSpec((_R, d), lambda i: (i, 0)),
            pl.BlockSpec((_NUM_EMB, d), lambda i: (0, 0)),
        ],
        out_specs=(
            pl.BlockSpec((_R, d), lambda i: (i, 0)),
            pl.BlockSpec((_R, _NUM_EMB), lambda i: (i, 0)),
            pl.BlockSpec((1, 1, _R), lambda i: (i, 0, 0)),
            pl.BlockSpec((1, 1), lambda i: (0, 0)),
        ),
        out_shape=out_types,
        scratch_shapes=[
            pltpu.VMEM((1, _NUM_EMB), jnp.float32),
            pltpu.VMEM((1, _NUM_EMB), jnp.float32),
            pltpu.VMEM((1, 1), jnp.float32),
        ],
    )(flat, W)
    return (q.reshape(b, l, d), loss.reshape(()),
            enc.reshape(b, l, _NUM_EMB), idx.reshape(b, l))


# speedup vs baseline: 1.2026x; 1.0497x over previous
import jax
import jax.numpy as jnp
from jax.experimental import pallas as pl
from jax.experimental.pallas import tpu as pltpu

_NUM_EMB = 1024
_DIM = 64
_N = 8 * 576
_R = 4608


def _vq_kernel(x_ref, w_ref, q_ref, enc_ref, idx_ref, loss_ref,
               w2_acc, counts_acc, mse_acc):
    i = pl.program_id(0)
    nsteps = pl.num_programs(0)

    xb = x_ref[...]
    w = w_ref[...]

    @pl.when(i == 0)
    def _w2():
        w2_acc[...] = jnp.sum(w * w, axis=1)[None, :]

    x2 = jnp.sum(xb * xb, axis=1, keepdims=True)
    xw = jax.lax.dot_general(xb, w, (((1,), (1,)), ((), ())),
                             preferred_element_type=jnp.float32)
    d2 = x2 + w2_acc[...] - 2.0 * xw

    idx = jnp.argmin(d2, axis=1).astype(jnp.int32)
    idx_ref[0, 0] = idx

    cols = jax.lax.broadcasted_iota(jnp.int32, d2.shape, 1)
    one_hot = (cols == idx[:, None]).astype(jnp.float32)
    enc_ref[...] = one_hot

    q = jax.lax.dot_general(one_hot, w, (((1,), (0,)), ((), ())),
                            preferred_element_type=jnp.float32)
    q_ref[...] = q

    diff = q - xb
    mse_part = jnp.sum(diff * diff).reshape(1, 1)
    counts_part = jnp.sum(one_hot, axis=0, keepdims=True)

    @pl.when(i == 0)
    def _init():
        counts_acc[...] = counts_part
        mse_acc[...] = mse_part

    @pl.when(i != 0)
    def _acc():
        counts_acc[...] += counts_part
        mse_acc[...] += mse_part

    @pl.when(i == nsteps - 1)
    def _finalize():
        usage = counts_acc[...] * (1.0 / _N)
        du = usage - (1.0 / _NUM_EMB)
        usage_loss = jnp.sum(du * du) * (1.0 / _NUM_EMB)
        mse = mse_acc[...] * (1.0 / (_N * _DIM))
        loss_ref[...] = 1.25 * mse + 0.1 * usage_loss


@jax.jit
def kernel(x, W):
    b, l, d = x.shape
    n = b * l
    flat = x.reshape(n, d)
    grid = (n // _R,)
    out_types = (
        jax.ShapeDtypeStruct((n, d), jnp.float32),
        jax.ShapeDtypeStruct((n, _NUM_EMB), jnp.float32),
        jax.ShapeDtypeStruct((n // _R, 1, _R), jnp.int32),
        jax.ShapeDtypeStruct((1, 1), jnp.float32),
    )
    q, enc, idx, loss = pl.pallas_call(
        _vq_kernel,
        grid=grid,
        in_specs=[
            pl.BlockSpec((_R, d), lambda i: (i, 0)),
            pl.BlockSpec((_NUM_EMB, d), lambda i: (0, 0)),
        ],
        out_specs=(
            pl.BlockSpec((_R, d), lambda i: (i, 0)),
            pl.BlockSpec((_R, _NUM_EMB), lambda i: (i, 0)),
            pl.BlockSpec((1, 1, _R), lambda i: (i, 0, 0)),
            pl.BlockSpec((1, 1), lambda i: (0, 0)),
        ),
        out_shape=out_types,
        scratch_shapes=[
            pltpu.VMEM((1, _NUM_EMB), jnp.float32),
            pltpu.VMEM((1, _NUM_EMB), jnp.float32),
            pltpu.VMEM((1, 1), jnp.float32),
        ],
    )(flat, W)
    return (q.reshape(b, l, d), loss.reshape(()),
            enc.reshape(b, l, _NUM_EMB), idx.reshape(b, l))
